# baseline (speedup 1.0000x reference)
import functools
import jax
import jax.numpy as jnp
from jax import lax
from jax.experimental import pallas as pl
from jax.experimental.pallas import tpu as pltpu
from jax.experimental.pallas import tpu_sc as plsc

D = 768
I = 1536
H = 12
P = 128
N = 64
DCONV = 4
CH = 256
E = 8
TOPK = 2
FF = 768
EPS = 1e-6
RM = 0.22
CONV_DIM = I + 2 * N
PROJ = 2 * I + 2 * N + H
PROJ_PAD = 3328
BK = 256
HI = jax.lax.Precision.HIGHEST


def _rmsnorm(x, w):
    v = jnp.mean(x * x, axis=-1, keepdims=True)
    return x * jax.lax.rsqrt(v + EPS) * w


def _silu(x):
    return x * jax.nn.sigmoid(x)


def _mixer_body(hs_ref, res_ref, iln_ref, pln_ref, wproj_ref, convw_ref,
                convb_ref, alog_ref, dtb_ref, dpar_ref, snw_ref, wout_ref,
                rw_ref,
                res2_ref, tok_ref, gate_ref, pos_ref, be_ref, act_ref,
                conv_carry, state_carry, cnt_carry, expsc, ranksc,
                wp_bf, wo_bf):
    c = pl.program_id(0)
    nc = pl.num_programs(0)
    S = nc * CH
    NB = (TOPK * S + E * BK) // BK

    @pl.when(c == 0)
    def _init():
        conv_carry[...] = jnp.zeros_like(conv_carry)
        state_carry[...] = jnp.zeros_like(state_carry)
        cnt_carry[...] = jnp.zeros_like(cnt_carry)
        wp_bf[...] = wproj_ref[...].astype(jnp.bfloat16)
        wo_bf[...] = wout_ref[...].astype(jnp.bfloat16)

    x = hs_ref[...] + res_ref[...]
    h = _rmsnorm(x, iln_ref[...])

    zxbcdt = jax.lax.dot_general(
        h.astype(jnp.bfloat16), wp_bf[...],
        (((1,), (1,)), ((), ())),
        preferred_element_type=jnp.float32)
    z = zxbcdt[:, :I]
    xBC_r = zxbcdt[:, I:I + CONV_DIM]
    dt_raw = zxbcdt[:, I + CONV_DIM:I + CONV_DIM + H]

    tail = conv_carry[...]
    xfull = jnp.concatenate([tail[DCONV + 1:, :], xBC_r], axis=0)
    acc = jnp.broadcast_to(convb_ref[...], (CH, CONV_DIM))
    for k in range(DCONV):
        acc = acc + xfull[k:k + CH, :] * convw_ref[k:k + 1, :]
    conv_carry[...] = xBC_r[CH - 8:, :]
    xBC = _silu(acc)

    xin = xBC[:, :I]
    Bm = xBC[:, I:I + N]
    Cm = xBC[:, I + N:]

    dt = jax.nn.softplus(dt_raw + dtb_ref[...])
    A = -jnp.exp(alog_ref[...])
    dA = dt * A

    rid = lax.broadcasted_iota(jnp.int32, (CH, CH), 0)
    cid = lax.broadcasted_iota(jnp.int32, (CH, CH), 1)
    tril_i = (rid >= cid).astype(jnp.float32)
    tril_e = (rid > cid).astype(jnp.float32)
    A_cs = jax.lax.dot_general(tril_i, dA, (((1,), (0,)), ((), ())),
                               precision=HI,
                               preferred_element_type=jnp.float32)

    CBT = jax.lax.dot_general(Cm, Bm, (((1,), (1,)), ((), ())),
                              precision=HI,
                              preferred_element_type=jnp.float32)
    mask = rid >= cid

    y_parts = []
    for hh in range(H):
        a_h = A_cs[:, hh:hh + 1]
        diff = jnp.where(mask, a_h - a_h.reshape(1, CH), -1e9)
        L = jnp.exp(diff)
        xin_h = xin[:, hh * P:(hh + 1) * P]
        X_h = xin_h * dt[:, hh:hh + 1]
        Yd = jax.lax.dot_general(
            (CBT * L).astype(jnp.bfloat16), X_h.astype(jnp.bfloat16),
            (((1,), (0,)), ((), ())), preferred_element_type=jnp.float32)
        a_sum = A_cs[CH - 1:CH, hh:hh + 1]
        decay = jnp.exp(a_sum - a_h)
        S_old = state_carry[hh]
        chunk_state = jax.lax.dot_general(
            Bm.astype(jnp.bfloat16), (X_h * decay).astype(jnp.bfloat16),
            (((0,), (0,)), ((), ())), preferred_element_type=jnp.float32)
        state_carry[hh] = jnp.exp(a_sum) * S_old + chunk_state
        Yo = jax.lax.dot_general(
            Cm.astype(jnp.bfloat16), S_old.astype(jnp.bfloat16),
            (((1,), (0,)), ((), ())), preferred_element_type=jnp.float32)
        Yo = Yo * jnp.exp(a_h)
        y_parts.append(Yd + Yo + dpar_ref[0, hh] * xin_h)
    y = jnp.concatenate(y_parts, axis=1)

    y = _rmsnorm(y * _silu(z), snw_ref[...])
    attn = jax.lax.dot_general(
        y.astype(jnp.bfloat16), wo_bf[...],
        (((1,), (1,)), ((), ())), preferred_element_type=jnp.float32)

    h1 = x + attn * RM
    res2_ref[...] = h1
    h2 = _rmsnorm(h1, pln_ref[...])
    tok_ref[...] = h2.astype(jnp.bfloat16)

    logits = jax.lax.dot_general(h2, rw_ref[...], (((1,), (1,)), ((), ())),
                                 precision=HI,
                                 preferred_element_type=jnp.float32)
    eid = lax.broadcasted_iota(jnp.int32, (CH, E), 1)
    m1 = jnp.max(logits, axis=1, keepdims=True)
    i1 = jnp.min(jnp.where(logits == m1, eid, E), axis=1, keepdims=True)
    lg2 = jnp.where(eid == i1, -jnp.inf, logits)
    m2 = jnp.max(lg2, axis=1, keepdims=True)
    i2 = jnp.min(jnp.where(lg2 == m2, eid, E), axis=1, keepdims=True)
    ex = jnp.exp(m2 - m1)
    g1 = 1.0 / (1.0 + ex)
    g2 = ex * g1

    oh1 = (eid == i1).astype(jnp.float32)
    oh2 = (eid == i2).astype(jnp.float32)
    cnts = cnt_carry[...]
    cum1 = jax.lax.dot_general(tril_e, oh1, (((1,), (0,)), ((), ())),
                               precision=HI,
                               preferred_element_type=jnp.float32) + cnts
    rank1 = jnp.sum(cum1 * oh1, axis=1, keepdims=True)
    cnts = cnts + jnp.sum(oh1, axis=0, keepdims=True)
    cum2 = jax.lax.dot_general(tril_e, oh2, (((1,), (0,)), ((), ())),
                               precision=HI,
                               preferred_element_type=jnp.float32) + cnts
    rank2 = jnp.sum(cum2 * oh2, axis=1, keepdims=True)
    cnts = cnts + jnp.sum(oh2, axis=0, keepdims=True)
    cnt_carry[...] = cnts

    gate_ref[...] = (jnp.concatenate([g1, g2], axis=1) * RM).T
    expsc[:, pl.ds(c * CH, CH)] = jnp.concatenate(
        [i1.astype(jnp.float32), i2.astype(jnp.float32)], axis=1).T
    ranksc[:, pl.ds(c * CH, CH)] = jnp.concatenate([rank1, rank2], axis=1).T

    @pl.when(c == nc - 1)
    def _fin():
        pc = jnp.floor((cnts + (BK - 1.0)) * (1.0 / BK)) * BK
        lid = lax.broadcasted_iota(jnp.int32, (1, E), 1)
        cum = jnp.zeros((1, E), jnp.float32)
        for k in range(E):
            cum = cum + jnp.where(lid >= k, pc[0:1, k:k + 1], 0.0)
        poff = cum - pc
        ev = expsc[...]
        pos = ranksc[...]
        for k in range(E):
            pos = pos + jnp.where(ev == k, poff[0:1, k:k + 1], 0.0)
        pos_ref[...] = pos.astype(jnp.int32)
        bstart = lax.broadcasted_iota(
            jnp.int32, (1, 128), 1).astype(jnp.float32) * BK
        bacc = jnp.zeros((1, 128), jnp.float32)
        for k in range(E):
            bacc = bacc + (bstart >= cum[0:1, k:k + 1]).astype(jnp.float32)
        be_ref[...] = jnp.minimum(bacc, E - 1.0).astype(jnp.int32)[:, :NB]
        act_ref[...] = (bstart < cum[0:1, E - 1:E]).astype(jnp.int32)[:, :NB]


def _run_mixer(hs, res, iln, pln, wproj_p, convwT, convb, alog, dtb, dpar,
               snw, wout, rw, S, interpret=False):
    nc = S // CH
    NB = (TOPK * S + E * BK) // BK
    grid = (nc,)
    full = lambda *shape: pl.BlockSpec(shape, lambda c: tuple(0 for _ in shape))
    seq = pl.BlockSpec((CH, D), lambda c: (c, 0))
    out_shapes = [
        jax.ShapeDtypeStruct((S, D), jnp.float32),
        jax.ShapeDtypeStruct((S, D), jnp.bfloat16),
        jax.ShapeDtypeStruct((2, S), jnp.float32),
        jax.ShapeDtypeStruct((2, S), jnp.int32),
        jax.ShapeDtypeStruct((1, NB), jnp.int32),
        jax.ShapeDtypeStruct((1, NB), jnp.int32),
    ]
    out_specs = [
        seq, seq,
        pl.BlockSpec((2, CH), lambda c: (0, c)),
        full(2, S),
        full(1, NB),
        full(1, NB),
    ]
    in_specs = [
        seq, seq,
        full(1, D), full(1, D),
        full(PROJ, D),
        full(DCONV, CONV_DIM), full(1, CONV_DIM),
        full(1, H), full(1, H), full(1, H),
        full(1, I),
        full(D, I),
        full(E, D),
    ]
    return pl.pallas_call(
        _mixer_body,
        grid=grid,
        in_specs=in_specs,
        out_specs=out_specs,
        out_shape=out_shapes,
        scratch_shapes=[
            pltpu.VMEM((8, CONV_DIM), jnp.float32),
            pltpu.VMEM((H, N, P), jnp.float32),
            pltpu.VMEM((1, E), jnp.float32),
            pltpu.VMEM((2, S), jnp.float32),
            pltpu.VMEM((2, S), jnp.float32),
            pltpu.VMEM((PROJ, D), jnp.bfloat16),
            pltpu.VMEM((D, I), jnp.bfloat16),
        ],
        compiler_params=pltpu.CompilerParams(
            dimension_semantics=("arbitrary",)),
        interpret=interpret,
    )(hs, res, iln, pln, wproj_p, convwT, convb, alog, dtb, dpar, snw,
      wout, rw)


def _make_sc_dispatch(S, NPAD):
    NPAIR = TOPK * S
    NW = 32
    PW = NPAIR // NW

    mesh = plsc.VectorSubcoreMesh(core_axis_name="c", subcore_axis_name="s")

    DW = D // 2

    @functools.partial(
        pl.kernel,
        out_type=jax.ShapeDtypeStruct((NPAD, DW), jnp.int32),
        mesh=mesh,
        scratch_types=[
            pltpu.VMEM((PW,), jnp.int32),
            pltpu.VMEM((PW, DW), jnp.int32),
            pltpu.SemaphoreType.DMA,
        ],
    )
    def sc_dispatch(pos_hbm, t_hbm, xs_hbm, pos_v, rows_v, sem):
        wid = lax.axis_index("s") * 2 + lax.axis_index("c")
        base = wid * PW
        tokbase = base % S
        pltpu.sync_copy(pos_hbm.at[pl.ds(base, PW)], pos_v)
        pltpu.sync_copy(t_hbm.at[pl.ds(tokbase, PW)], rows_v)
        pltpu.async_copy(rows_v, xs_hbm.at[pos_v], sem).wait()

    return sc_dispatch


def _make_sc_combine(S, NPAD):
    NW = 32
    TW = S // NW
    GCH = min(TW, 128)

    mesh = plsc.VectorSubcoreMesh(core_axis_name="c", subcore_axis_name="s")

    @functools.partial(
        pl.kernel,
        out_type=[
            jax.ShapeDtypeStruct((S, D // 2), jnp.int32),
            jax.ShapeDtypeStruct((S, D // 2), jnp.int32),
        ],
        mesh=mesh,
        scratch_types=[
            pltpu.VMEM((GCH,), jnp.int32),
            pltpu.VMEM((GCH, D // 2), jnp.int32),
            pltpu.SemaphoreType.DMA,
        ],
    )
    def sc_combine(pos_hbm, y_hbm, g0_hbm, g1_hbm, idx_v, rows_v, sem):
        wid = lax.axis_index("s") * 2 + lax.axis_index("c")
        for slot in range(TOPK):
            for sub in range(TW // GCH):
                base = wid * TW + sub * GCH
                pltpu.sync_copy(pos_hbm.at[pl.ds(slot * S + base, GCH)], idx_v)
                pltpu.async_copy(y_hbm.at[idx_v], rows_v, sem).wait()
                out = g0_hbm if slot == 0 else g1_hbm
                pltpu.sync_copy(rows_v, out.at[pl.ds(base, GCH)])

    return sc_combine


def _moe_body(be_ref, act_ref, x_ref, w13_ref, w2_ref, y_ref):
    i = pl.program_id(0)

    @pl.when(act_ref[i] == 1)
    def _():
        x = x_ref[...]
        g = jax.lax.dot_general(
            x, w13_ref[0].astype(jnp.bfloat16),
            (((1,), (1,)), ((), ())),
            preferred_element_type=jnp.float32)
        a = (_silu(g[:, :FF]) * g[:, FF:]).astype(jnp.bfloat16)
        y_ref[...] = jax.lax.dot_general(
            a, w2_ref[0].astype(jnp.bfloat16),
            (((1,), (1,)), ((), ())),
            preferred_element_type=jnp.float32).astype(jnp.bfloat16)


def _run_moe(be, act, x_sorted, w13, w2, NPAD, interpret=False):
    NB = NPAD // BK
    grid_spec = pltpu.PrefetchScalarGridSpec(
        num_scalar_prefetch=2,
        grid=(NB,),
        in_specs=[
            pl.BlockSpec((BK, D), lambda i, be, act: (i, 0)),
            pl.BlockSpec((1, 2 * FF, D), lambda i, be, act: (be[i], 0, 0)),
            pl.BlockSpec((1, D, FF), lambda i, be, act: (be[i], 0, 0)),
        ],
        out_specs=pl.BlockSpec((BK, D), lambda i, be, act: (i, 0)),
    )
    return pl.pallas_call(
        _moe_body,
        grid_spec=grid_spec,
        out_shape=jax.ShapeDtypeStruct((NPAD, D), jnp.bfloat16),
        compiler_params=pltpu.CompilerParams(
            dimension_semantics=("arbitrary",)),
        interpret=interpret,
    )(be, act, x_sorted, w13, w2)


def _add_body(r_ref, a_ref, b_ref, g_ref, o_ref):
    gc = g_ref[...].T
    o_ref[...] = (r_ref[...]
                  + a_ref[...].astype(jnp.float32) * gc[:, 0:1]
                  + b_ref[...].astype(jnp.float32) * gc[:, 1:2])


def _run_add(res2, g0, g1, gatef, S, interpret=False):
    seq = pl.BlockSpec((CH, D), lambda c: (c, 0))
    return pl.pallas_call(
        _add_body,
        grid=(S // CH,),
        in_specs=[seq, seq, seq, pl.BlockSpec((2, CH), lambda c: (0, c))],
        out_specs=seq,
        out_shape=jax.ShapeDtypeStruct((S, D), jnp.float32),
        interpret=interpret,
    )(res2, g0, g1, gatef)


def kernel(hidden_states, residual, input_ln_w, post_ln_w, in_proj_w, conv_w,
           conv_b, A_log, dt_bias, D_param, ssm_norm_w, out_proj_w,
           router_w, w13, w2):
    b, S, d = hidden_states.shape
    NPAIR = TOPK * S
    NPAD = NPAIR + E * BK
    NB = NPAD // BK
    hs = hidden_states.reshape(S, D)
    res = residual.reshape(S, D)
    convwT = conv_w.T
    r2 = lambda a: a.reshape(1, -1)

    res2, tok, gatef, posf, bef, actf = _run_mixer(
        hs, res, r2(input_ln_w), r2(post_ln_w), in_proj_w, convwT, r2(conv_b),
        r2(A_log), r2(dt_bias), r2(D_param), r2(ssm_norm_w),
        out_proj_w, router_w, S)

    be = bef.reshape(NB)
    act = actf.reshape(NB)
    pos = posf.reshape(NPAIR)

    bits = lambda a: lax.bitcast_convert_type(
        a.reshape(a.shape[0], D // 2, 2), jnp.int32)
    unbits = lambda a: lax.bitcast_convert_type(
        a, jnp.bfloat16).reshape(a.shape[0], D)

    sc_dispatch = _make_sc_dispatch(S, NPAD)
    x_sorted = unbits(sc_dispatch(pos, bits(tok)))

    y_sorted = _run_moe(be, act, x_sorted, w13, w2, NPAD)

    sc_combine = _make_sc_combine(S, NPAD)
    g0, g1 = sc_combine(pos, bits(y_sorted))

    h = _run_add(res2, unbits(g0), unbits(g1), gatef, S)
    return h.reshape(b, S, d), res2.reshape(b, S, d)


# speedup vs baseline: 2.6944x; 2.6944x over previous
import functools
import jax
import jax.numpy as jnp
from jax import lax
from jax.experimental import pallas as pl
from jax.experimental.pallas import tpu as pltpu
from jax.experimental.pallas import tpu_sc as plsc

D = 768
I = 1536
H = 12
P = 128
N = 64
DCONV = 4
CH = 256
E = 8
TOPK = 2
FF = 768
EPS = 1e-6
RM = 0.22
CONV_DIM = I + 2 * N
PROJ = 2 * I + 2 * N + H
PROJ_PAD = 3328
BK = 256
HI = jax.lax.Precision.HIGHEST


def _rmsnorm(x, w):
    v = jnp.mean(x * x, axis=-1, keepdims=True)
    return x * jax.lax.rsqrt(v + EPS) * w


def _silu(x):
    return x * jax.nn.sigmoid(x)


def _mixer_body(hs_ref, res_ref, iln_ref, pln_ref, wproj_ref, convw_ref,
                convb_ref, alog_ref, dtb_ref, dpar_ref, snw_ref, wout_ref,
                rw_ref,
                res2_ref, tok_ref, gate_ref, pos_ref, be_ref, act_ref,
                conv_carry, state_carry, cnt_carry, expsc, ranksc,
                wp_bf, wo_bf):
    c = pl.program_id(0)
    nc = pl.num_programs(0)
    S = nc * CH
    NB = (TOPK * S + E * BK) // BK

    @pl.when(c == 0)
    def _init():
        conv_carry[...] = jnp.zeros_like(conv_carry)
        state_carry[...] = jnp.zeros_like(state_carry)
        cnt_carry[...] = jnp.zeros_like(cnt_carry)
        wp_bf[...] = wproj_ref[...].astype(jnp.bfloat16)
        wo_bf[...] = wout_ref[...].astype(jnp.bfloat16)

    x = hs_ref[...] + res_ref[...]
    h = _rmsnorm(x, iln_ref[...])

    zxbcdt = jax.lax.dot_general(
        h.astype(jnp.bfloat16), wp_bf[...],
        (((1,), (1,)), ((), ())),
        preferred_element_type=jnp.float32)
    z = zxbcdt[:, :I]
    xBC_r = zxbcdt[:, I:I + CONV_DIM]
    dt_raw = zxbcdt[:, I + CONV_DIM:I + CONV_DIM + H]

    tail = conv_carry[...]
    xfull = jnp.concatenate([tail[DCONV + 1:, :], xBC_r], axis=0)
    acc = jnp.broadcast_to(convb_ref[...], (CH, CONV_DIM))
    for k in range(DCONV):
        acc = acc + xfull[k:k + CH, :] * convw_ref[k:k + 1, :]
    conv_carry[...] = xBC_r[CH - 8:, :]
    xBC = _silu(acc)

    xin = xBC[:, :I]
    Bm = xBC[:, I:I + N]
    Cm = xBC[:, I + N:]

    dt = jax.nn.softplus(dt_raw + dtb_ref[...])
    A = -jnp.exp(alog_ref[...])
    dA = dt * A

    rid = lax.broadcasted_iota(jnp.int32, (CH, CH), 0)
    cid = lax.broadcasted_iota(jnp.int32, (CH, CH), 1)
    tril_i = (rid >= cid).astype(jnp.float32)
    tril_e = (rid > cid).astype(jnp.float32)
    A_cs = jax.lax.dot_general(tril_i, dA, (((1,), (0,)), ((), ())),
                               precision=HI,
                               preferred_element_type=jnp.float32)

    CBT = jax.lax.dot_general(Cm, Bm, (((1,), (1,)), ((), ())),
                              precision=HI,
                              preferred_element_type=jnp.float32)
    mask = rid >= cid

    y_parts = []
    for hh in range(H):
        a_h = A_cs[:, hh:hh + 1]
        diff = jnp.where(mask, a_h - a_h.reshape(1, CH), -1e9)
        L = jnp.exp(diff)
        xin_h = xin[:, hh * P:(hh + 1) * P]
        X_h = xin_h * dt[:, hh:hh + 1]
        Yd = jax.lax.dot_general(
            (CBT * L).astype(jnp.bfloat16), X_h.astype(jnp.bfloat16),
            (((1,), (0,)), ((), ())), preferred_element_type=jnp.float32)
        a_sum = A_cs[CH - 1:CH, hh:hh + 1]
        decay = jnp.exp(a_sum - a_h)
        S_old = state_carry[hh]
        chunk_state = jax.lax.dot_general(
            Bm.astype(jnp.bfloat16), (X_h * decay).astype(jnp.bfloat16),
            (((0,), (0,)), ((), ())), preferred_element_type=jnp.float32)
        state_carry[hh] = jnp.exp(a_sum) * S_old + chunk_state
        Yo = jax.lax.dot_general(
            Cm.astype(jnp.bfloat16), S_old.astype(jnp.bfloat16),
            (((1,), (0,)), ((), ())), preferred_element_type=jnp.float32)
        Yo = Yo * jnp.exp(a_h)
        y_parts.append(Yd + Yo + dpar_ref[0, hh] * xin_h)
    y = jnp.concatenate(y_parts, axis=1)

    y = _rmsnorm(y * _silu(z), snw_ref[...])
    attn = jax.lax.dot_general(
        y.astype(jnp.bfloat16), wo_bf[...],
        (((1,), (1,)), ((), ())), preferred_element_type=jnp.float32)

    h1 = x + attn * RM
    res2_ref[...] = h1
    h2 = _rmsnorm(h1, pln_ref[...])
    tok_ref[...] = h2

    logits = jax.lax.dot_general(h2, rw_ref[...], (((1,), (1,)), ((), ())),
                                 precision=HI,
                                 preferred_element_type=jnp.float32)
    eid = lax.broadcasted_iota(jnp.int32, (CH, E), 1)
    m1 = jnp.max(logits, axis=1, keepdims=True)
    i1 = jnp.min(jnp.where(logits == m1, eid, E), axis=1, keepdims=True)
    lg2 = jnp.where(eid == i1, -jnp.inf, logits)
    m2 = jnp.max(lg2, axis=1, keepdims=True)
    i2 = jnp.min(jnp.where(lg2 == m2, eid, E), axis=1, keepdims=True)
    ex = jnp.exp(m2 - m1)
    g1 = 1.0 / (1.0 + ex)
    g2 = ex * g1

    oh1 = (eid == i1).astype(jnp.float32)
    oh2 = (eid == i2).astype(jnp.float32)
    cnts = cnt_carry[...]
    cum1 = jax.lax.dot_general(tril_e, oh1, (((1,), (0,)), ((), ())),
                               precision=HI,
                               preferred_element_type=jnp.float32) + cnts
    rank1 = jnp.sum(cum1 * oh1, axis=1, keepdims=True)
    cnts = cnts + jnp.sum(oh1, axis=0, keepdims=True)
    cum2 = jax.lax.dot_general(tril_e, oh2, (((1,), (0,)), ((), ())),
                               precision=HI,
                               preferred_element_type=jnp.float32) + cnts
    rank2 = jnp.sum(cum2 * oh2, axis=1, keepdims=True)
    cnts = cnts + jnp.sum(oh2, axis=0, keepdims=True)
    cnt_carry[...] = cnts

    gate_ref[...] = (jnp.concatenate([g1, g2], axis=1) * RM).T
    expsc[:, pl.ds(c * CH, CH)] = jnp.concatenate(
        [i1.astype(jnp.float32), i2.astype(jnp.float32)], axis=1).T
    ranksc[:, pl.ds(c * CH, CH)] = jnp.concatenate([rank1, rank2], axis=1).T

    @pl.when(c == nc - 1)
    def _fin():
        pc = jnp.floor((cnts + (BK - 1.0)) * (1.0 / BK)) * BK
        lid = lax.broadcasted_iota(jnp.int32, (1, E), 1)
        cum = jnp.zeros((1, E), jnp.float32)
        for k in range(E):
            cum = cum + jnp.where(lid >= k, pc[0:1, k:k + 1], 0.0)
        poff = cum - pc
        ev = expsc[...]
        pos = ranksc[...]
        for k in range(E):
            pos = pos + jnp.where(ev == k, poff[0:1, k:k + 1], 0.0)
        pos_ref[...] = pos.astype(jnp.int32)
        bstart = lax.broadcasted_iota(
            jnp.int32, (1, 128), 1).astype(jnp.float32) * BK
        bacc = jnp.zeros((1, 128), jnp.float32)
        for k in range(E):
            bacc = bacc + (bstart >= cum[0:1, k:k + 1]).astype(jnp.float32)
        be_ref[...] = jnp.minimum(bacc, E - 1.0).astype(jnp.int32)[:, :NB]
        act_ref[...] = (bstart < cum[0:1, E - 1:E]).astype(jnp.int32)[:, :NB]


def _run_mixer(hs, res, iln, pln, wproj_p, convwT, convb, alog, dtb, dpar,
               snw, wout, rw, S, interpret=False):
    nc = S // CH
    NB = (TOPK * S + E * BK) // BK
    grid = (nc,)
    full = lambda *shape: pl.BlockSpec(shape, lambda c: tuple(0 for _ in shape))
    seq = pl.BlockSpec((CH, D), lambda c: (c, 0))
    out_shapes = [
        jax.ShapeDtypeStruct((S, D), jnp.float32),
        jax.ShapeDtypeStruct((S, D), jnp.float32),
        jax.ShapeDtypeStruct((2, S), jnp.float32),
        jax.ShapeDtypeStruct((2, S), jnp.int32),
        jax.ShapeDtypeStruct((1, NB), jnp.int32),
        jax.ShapeDtypeStruct((1, NB), jnp.int32),
    ]
    out_specs = [
        seq, seq,
        pl.BlockSpec((2, CH), lambda c: (0, c)),
        full(2, S),
        full(1, NB),
        full(1, NB),
    ]
    in_specs = [
        seq, seq,
        full(1, D), full(1, D),
        full(PROJ, D),
        full(DCONV, CONV_DIM), full(1, CONV_DIM),
        full(1, H), full(1, H), full(1, H),
        full(1, I),
        full(D, I),
        full(E, D),
    ]
    return pl.pallas_call(
        _mixer_body,
        grid=grid,
        in_specs=in_specs,
        out_specs=out_specs,
        out_shape=out_shapes,
        scratch_shapes=[
            pltpu.VMEM((8, CONV_DIM), jnp.float32),
            pltpu.VMEM((H, N, P), jnp.float32),
            pltpu.VMEM((1, E), jnp.float32),
            pltpu.VMEM((2, S), jnp.float32),
            pltpu.VMEM((2, S), jnp.float32),
            pltpu.VMEM((PROJ, D), jnp.bfloat16),
            pltpu.VMEM((D, I), jnp.bfloat16),
        ],
        compiler_params=pltpu.CompilerParams(
            dimension_semantics=("arbitrary",)),
        interpret=interpret,
    )(hs, res, iln, pln, wproj_p, convwT, convb, alog, dtb, dpar, snw,
      wout, rw)


def _make_sc_dispatch(S, NPAD):
    NPAIR = TOPK * S
    NW = 32
    PW = NPAIR // NW

    mesh = plsc.VectorSubcoreMesh(core_axis_name="c", subcore_axis_name="s")

    @functools.partial(
        pl.kernel,
        out_type=jax.ShapeDtypeStruct((NPAD, D), jnp.float32),
        mesh=mesh,
        scratch_types=[
            pltpu.VMEM((PW,), jnp.int32),
            pltpu.VMEM((PW, D), jnp.float32),
            pltpu.SemaphoreType.DMA,
        ],
    )
    def sc_dispatch(pos_hbm, t_hbm, xs_hbm, pos_v, rows_v, sem):
        wid = lax.axis_index("s") * 2 + lax.axis_index("c")
        base = wid * PW
        tokbase = base % S
        pltpu.sync_copy(pos_hbm.at[pl.ds(base, PW)], pos_v)
        pltpu.sync_copy(t_hbm.at[pl.ds(tokbase, PW)], rows_v)
        pltpu.async_copy(rows_v, xs_hbm.at[pos_v], sem).wait()

    return sc_dispatch


def _make_sc_combine(S, NPAD):
    NW = 32
    TW = S // NW
    GCH = min(TW, 128)

    mesh = plsc.VectorSubcoreMesh(core_axis_name="c", subcore_axis_name="s")

    @functools.partial(
        pl.kernel,
        out_type=[
            jax.ShapeDtypeStruct((S, D), jnp.float32),
            jax.ShapeDtypeStruct((S, D), jnp.float32),
        ],
        mesh=mesh,
        scratch_types=[
            pltpu.VMEM((GCH,), jnp.int32),
            pltpu.VMEM((GCH, D), jnp.float32),
            pltpu.SemaphoreType.DMA,
        ],
    )
    def sc_combine(pos_hbm, y_hbm, g0_hbm, g1_hbm, idx_v, rows_v, sem):
        wid = lax.axis_index("s") * 2 + lax.axis_index("c")
        for slot in range(TOPK):
            for sub in range(TW // GCH):
                base = wid * TW + sub * GCH
                pltpu.sync_copy(pos_hbm.at[pl.ds(slot * S + base, GCH)], idx_v)
                pltpu.async_copy(y_hbm.at[idx_v], rows_v, sem).wait()
                out = g0_hbm if slot == 0 else g1_hbm
                pltpu.sync_copy(rows_v, out.at[pl.ds(base, GCH)])

    return sc_combine


def _moe_body(be_ref, act_ref, x_ref, w13_ref, w2_ref, y_ref):
    i = pl.program_id(0)

    @pl.when(act_ref[i] == 1)
    def _():
        x = x_ref[...].astype(jnp.bfloat16)
        g = jax.lax.dot_general(
            x, w13_ref[0].astype(jnp.bfloat16),
            (((1,), (1,)), ((), ())),
            preferred_element_type=jnp.float32)
        a = (_silu(g[:, :FF]) * g[:, FF:]).astype(jnp.bfloat16)
        y_ref[...] = jax.lax.dot_general(
            a, w2_ref[0].astype(jnp.bfloat16),
            (((1,), (1,)), ((), ())),
            preferred_element_type=jnp.float32)


def _run_moe(be, act, x_sorted, w13, w2, NPAD, interpret=False):
    NB = NPAD // BK
    grid_spec = pltpu.PrefetchScalarGridSpec(
        num_scalar_prefetch=2,
        grid=(NB,),
        in_specs=[
            pl.BlockSpec((BK, D), lambda i, be, act: (i, 0)),
            pl.BlockSpec((1, 2 * FF, D), lambda i, be, act: (be[i], 0, 0)),
            pl.BlockSpec((1, D, FF), lambda i, be, act: (be[i], 0, 0)),
        ],
        out_specs=pl.BlockSpec((BK, D), lambda i, be, act: (i, 0)),
    )
    return pl.pallas_call(
        _moe_body,
        grid_spec=grid_spec,
        out_shape=jax.ShapeDtypeStruct((NPAD, D), jnp.float32),
        compiler_params=pltpu.CompilerParams(
            dimension_semantics=("arbitrary",)),
        interpret=interpret,
    )(be, act, x_sorted, w13, w2)


def _add_body(r_ref, a_ref, b_ref, g_ref, o_ref):
    gc = g_ref[...].T
    o_ref[...] = (r_ref[...] + a_ref[...] * gc[:, 0:1]
                  + b_ref[...] * gc[:, 1:2])


def _run_add(res2, g0, g1, gatef, S, interpret=False):
    seq = pl.BlockSpec((CH, D), lambda c: (c, 0))
    return pl.pallas_call(
        _add_body,
        grid=(S // CH,),
        in_specs=[seq, seq, seq, pl.BlockSpec((2, CH), lambda c: (0, c))],
        out_specs=seq,
        out_shape=jax.ShapeDtypeStruct((S, D), jnp.float32),
        interpret=interpret,
    )(res2, g0, g1, gatef)


def kernel(hidden_states, residual, input_ln_w, post_ln_w, in_proj_w, conv_w,
           conv_b, A_log, dt_bias, D_param, ssm_norm_w, out_proj_w,
           router_w, w13, w2):
    b, S, d = hidden_states.shape
    NPAIR = TOPK * S
    NPAD = NPAIR + E * BK
    NB = NPAD // BK
    hs = hidden_states.reshape(S, D)
    res = residual.reshape(S, D)
    convwT = conv_w.T
    r2 = lambda a: a.reshape(1, -1)

    res2, tok, gatef, posf, bef, actf = _run_mixer(
        hs, res, r2(input_ln_w), r2(post_ln_w), in_proj_w, convwT, r2(conv_b),
        r2(A_log), r2(dt_bias), r2(D_param), r2(ssm_norm_w),
        out_proj_w, router_w, S)

    be = bef.reshape(NB)
    act = actf.reshape(NB)
    pos = posf.reshape(NPAIR)

    sc_dispatch = _make_sc_dispatch(S, NPAD)
    x_sorted = sc_dispatch(pos, tok)

    y_sorted = _run_moe(be, act, x_sorted, w13, w2, NPAD)

    sc_combine = _make_sc_combine(S, NPAD)
    g0, g1 = sc_combine(pos, y_sorted)

    h = _run_add(res2, g0, g1, gatef, S)
    return h.reshape(b, S, d), res2.reshape(b, S, d)
